# Initial kernel scaffold; baseline (speedup 1.0000x reference)
import jax
import jax.numpy as jnp
from jax.experimental import pallas as pl


def kernel(x, hw, hb, cw, cb, tw, tb):
    raise NotImplementedError("write your pallas kernel here")


# speedup vs baseline: 1.4918x; 1.4918x over previous
import functools

import jax
import jax.numpy as jnp
from jax.experimental import pallas as pl
from jax.experimental.pallas import tpu as pltpu

_BATCH = 2


def _conv3x3(load_rows, load_w, bias_vec, H, W, Cout, relu):
    acc = jnp.broadcast_to(
        bias_vec.astype(jnp.float32)[None, None, :], (H, W, Cout))
    for dy in range(3):
        rows = load_rows(dy)
        patch = jnp.concatenate(
            [rows[:, 0:W, :], rows[:, 1:W + 1, :], rows[:, 2:W + 2, :]],
            axis=-1)
        acc = acc + jax.lax.dot_general(
            patch, load_w(dy),
            dimension_numbers=(((2,), (0,)), ((), ())),
            preferred_element_type=jnp.float32)
    if relu:
        acc = jnp.maximum(acc, 0.0)
    return acc


def _upscaler_kernel(x_ref, hw_ref, hb_ref, cw_ref, cb_ref, tw_ref, tb_ref,
                     o_ref, hpad_ref, bufa_ref, bufb_ref, *,
                     Hin, Win, Hup, Wup, chan, size, depth):
    bf16 = jnp.bfloat16

    zero_row = jnp.zeros((_BATCH, 1, Wup + 2, size), bf16)
    zero_col = jnp.zeros((_BATCH, Hup + 2, 1, size), bf16)
    for buf in (bufa_ref, bufb_ref):
        buf[:, 0:1, :, :] = zero_row
        buf[:, Hup + 1:Hup + 2, :, :] = zero_row
        buf[:, :, 0:1, :] = zero_col
        buf[:, :, Wup + 1:Wup + 2, :] = zero_col

    hpad_ref[...] = jnp.zeros((_BATCH, Hin + 2, Win + 2, chan), bf16)
    hpad_ref[:, 1:Hin + 1, 1:Win + 1, :] = x_ref[...].astype(bf16)
    for b in range(_BATCH):
        head = _conv3x3(
            lambda dy: hpad_ref[b, dy:dy + Hin, :, :],
            lambda dy: hw_ref[dy, :, :],
            hb_ref[0, :], Hin, Win, size, relu=True)
        up = jnp.repeat(jnp.repeat(head.astype(bf16), 2, axis=0), 2, axis=1)
        bufa_ref[b, 1:Hup + 1, 1:Wup + 1, :] = up

    def core_pair(i, carry):
        l0 = 2 * i
        l1 = l0 + 1
        for b in range(_BATCH):
            a0 = _conv3x3(
                lambda dy: bufa_ref[b, dy:dy + Hup, :, :],
                lambda dy: cw_ref[l0, dy, :, :],
                cb_ref[l0, 0, :], Hup, Wup, size, relu=True)
            bufb_ref[b, 1:Hup + 1, 1:Wup + 1, :] = a0.astype(bf16)
        for b in range(_BATCH):
            a1 = _conv3x3(
                lambda dy: bufb_ref[b, dy:dy + Hup, :, :],
                lambda dy: cw_ref[l1, dy, :, :],
                cb_ref[l1, 0, :], Hup, Wup, size, relu=True)
            bufa_ref[b, 1:Hup + 1, 1:Wup + 1, :] = a1.astype(bf16)
        return carry

    jax.lax.fori_loop(0, depth // 2, core_pair, 0)

    for b in range(_BATCH):
        tail = _conv3x3(
            lambda dy: bufa_ref[b, dy:dy + Hup, :, :],
            lambda dy: tw_ref[dy, :, :],
            tb_ref[0, :], Hup, Wup, chan, relu=False)
        o_ref[b, :, :, :] = tail.astype(o_ref.dtype)


def kernel(x, hw, hb, cw, cb, tw, tb):
    N, chan, Hin, Win = x.shape
    size = hw.shape[-1]
    depth = cw.shape[0]
    fac = 2
    Hup, Wup = Hin * fac, Win * fac
    bf16 = jnp.bfloat16

    xh = jnp.transpose(x, (0, 2, 3, 1))

    hw_f = hw.reshape(3, 3 * chan, size).astype(bf16)
    cw_f = cw.reshape(depth, 3, 3 * size, size).astype(bf16)
    tw_f = tw.reshape(3, 3 * size, chan).astype(bf16)
    hb_2 = hb.reshape(1, size)
    cb_3 = cb.reshape(depth, 1, size)
    tb_2 = tb.reshape(1, chan)

    kern = functools.partial(_upscaler_kernel, Hin=Hin, Win=Win, Hup=Hup,
                             Wup=Wup, chan=chan, size=size, depth=depth)

    out = pl.pallas_call(
        kern,
        out_shape=jax.ShapeDtypeStruct((N, Hup, Wup, chan), x.dtype),
        grid=(N // _BATCH,),
        in_specs=[
            pl.BlockSpec((_BATCH, Hin, Win, chan), lambda n: (n, 0, 0, 0)),
            pl.BlockSpec((3, 3 * chan, size), lambda n: (0, 0, 0)),
            pl.BlockSpec((1, size), lambda n: (0, 0)),
            pl.BlockSpec((depth, 3, 3 * size, size), lambda n: (0, 0, 0, 0)),
            pl.BlockSpec((depth, 1, size), lambda n: (0, 0, 0)),
            pl.BlockSpec((3, 3 * size, chan), lambda n: (0, 0, 0)),
            pl.BlockSpec((1, chan), lambda n: (0, 0)),
        ],
        out_specs=pl.BlockSpec((_BATCH, Hup, Wup, chan), lambda n: (n, 0, 0, 0)),
        scratch_shapes=[
            pltpu.VMEM((_BATCH, Hin + 2, Win + 2, chan), bf16),
            pltpu.VMEM((_BATCH, Hup + 2, Wup + 2, size), bf16),
            pltpu.VMEM((_BATCH, Hup + 2, Wup + 2, size), bf16),
        ],
        compiler_params=pltpu.CompilerParams(
            dimension_semantics=("parallel",),
            vmem_limit_bytes=64 * 1024 * 1024),
    )(xh, hw_f, hb_2, cw_f, cb_3, tw_f, tb_2)

    return jnp.transpose(out, (0, 3, 1, 2))


# speedup vs baseline: 2.3716x; 1.5898x over previous
import functools

import jax
import jax.numpy as jnp
from jax.experimental import pallas as pl
from jax.experimental.pallas import tpu as pltpu

_PAIRS = 2


def _conv3x3_packed(rows_all, w_ref_slices, bias_row, H, W, Cout, relu):
    acc = jnp.broadcast_to(bias_row[None, :, :], (H, W, Cout))
    for dy in range(3):
        rows = rows_all[dy:dy + H]
        patch = jnp.concatenate(
            [rows[:, 0:W, :], rows[:, 1:W + 1, :], rows[:, 2:W + 2, :]],
            axis=-1)
        acc = acc + jax.lax.dot_general(
            patch, w_ref_slices(dy),
            dimension_numbers=(((2,), (0,)), ((), ())),
            preferred_element_type=jnp.float32)
    if relu:
        acc = jnp.maximum(acc, 0.0)
    return acc


def _upscaler_kernel(x_ref, s2_ref, hw_ref, hb_ref, cw_ref, cb_ref, tw_ref,
                     tb_ref, o_ref, hpad_ref, bufa_ref, bufb_ref, *,
                     Hin, Win, Hup, Wup, cpk, spk, chan, depth):
    bf16 = jnp.bfloat16

    zero_row = jnp.zeros((_PAIRS, 1, Wup + 2, spk), bf16)
    zero_col = jnp.zeros((_PAIRS, Hup + 2, 1, spk), bf16)
    for buf in (bufa_ref, bufb_ref):
        buf[:, 0:1, :, :] = zero_row
        buf[:, Hup + 1:Hup + 2, :, :] = zero_row
        buf[:, :, 0:1, :] = zero_col
        buf[:, :, Wup + 1:Wup + 2, :] = zero_col

    hpad_ref[...] = jnp.zeros((_PAIRS, Hin + 2, Win + 2, cpk), bf16)
    hpad_ref[:, 1:Hin + 1, 1:Win + 1, :] = x_ref[...].astype(bf16)
    s2 = s2_ref[...]
    for p in range(_PAIRS):
        head = _conv3x3_packed(
            hpad_ref[p], lambda dy: hw_ref[dy, :, :],
            hb_ref[...], Hin, Win, spk, relu=True)
        up = jax.lax.dot_general(
            s2, head.astype(bf16).reshape(Hin * Win, spk),
            dimension_numbers=(((1,), (0,)), ((), ())),
            preferred_element_type=jnp.float32)
        bufa_ref[p, 1:Hup + 1, 1:Wup + 1, :] = (
            up.astype(bf16).reshape(Hup, Wup, spk))

    def core_pair(i, carry):
        l0 = 2 * i
        l1 = l0 + 1
        for p in range(_PAIRS):
            a0 = _conv3x3_packed(
                bufa_ref[p], lambda dy: cw_ref[l0, dy, :, :],
                cb_ref[l0], Hup, Wup, spk, relu=True)
            bufb_ref[p, 1:Hup + 1, 1:Wup + 1, :] = a0.astype(bf16)
        for p in range(_PAIRS):
            a1 = _conv3x3_packed(
                bufb_ref[p], lambda dy: cw_ref[l1, dy, :, :],
                cb_ref[l1], Hup, Wup, spk, relu=True)
            bufa_ref[p, 1:Hup + 1, 1:Wup + 1, :] = a1.astype(bf16)
        return carry

    jax.lax.fori_loop(0, depth // 2, core_pair, 0)

    for p in range(_PAIRS):
        tail = _conv3x3_packed(
            bufa_ref[p], lambda dy: tw_ref[dy, :, :],
            tb_ref[...], Hup, Wup, 2 * chan, relu=False)
        o_ref[p, :, :, :] = tail.astype(o_ref.dtype)


def _block_diag2(w):
    K, C = w.shape[-2], w.shape[-1]
    z = jnp.zeros(w.shape[:-2] + (K, C), w.dtype)
    top = jnp.concatenate([w, z], axis=-1)
    bot = jnp.concatenate([z, w], axis=-1)
    return jnp.concatenate([top, bot], axis=-2)


def _fold_packed(w):
    kH, kW, Cin, Cout = w.shape
    wbd = _block_diag2(w)
    return wbd.reshape(kH, kW * 2 * Cin, 2 * Cout)


def kernel(x, hw, hb, cw, cb, tw, tb):
    N, chan, Hin, Win = x.shape
    size = hw.shape[-1]
    depth = cw.shape[0]
    fac = 2
    Hup, Wup = Hin * fac, Win * fac
    cpk, spk = 2 * chan, 2 * size
    bf16 = jnp.bfloat16

    xh = jnp.transpose(x, (0, 2, 3, 1))
    xp = xh.reshape(N // 2, 2, Hin, Win, chan).transpose(
        0, 2, 3, 1, 4).reshape(N // 2, Hin, Win, cpk)

    pos = jnp.arange(Hup * Wup)
    src = (pos // Wup // fac) * Win + (pos % Wup) // fac
    s2 = jax.nn.one_hot(src, Hin * Win, dtype=bf16)

    hw_f = _fold_packed(hw).astype(bf16)
    cw_f = jax.vmap(_fold_packed)(cw).astype(bf16)
    tw_f = _fold_packed(tw).astype(bf16)
    hb_2 = jnp.tile(hb, 2).reshape(1, spk)
    cb_3 = jnp.tile(cb, (1, 2)).reshape(depth, 1, spk)
    tb_2 = jnp.tile(tb, 2).reshape(1, 2 * chan)

    kern = functools.partial(_upscaler_kernel, Hin=Hin, Win=Win, Hup=Hup,
                             Wup=Wup, cpk=cpk, spk=spk, chan=chan, depth=depth)

    out = pl.pallas_call(
        kern,
        out_shape=jax.ShapeDtypeStruct((N // 2, Hup, Wup, 2 * chan), x.dtype),
        grid=(N // 2 // _PAIRS,),
        in_specs=[
            pl.BlockSpec((_PAIRS, Hin, Win, cpk), lambda n: (n, 0, 0, 0)),
            pl.BlockSpec((Hup * Wup, Hin * Win), lambda n: (0, 0)),
            pl.BlockSpec((3, 2 * 3 * chan, spk), lambda n: (0, 0, 0)),
            pl.BlockSpec((1, spk), lambda n: (0, 0)),
            pl.BlockSpec((depth, 3, 2 * 3 * size, spk), lambda n: (0, 0, 0, 0)),
            pl.BlockSpec((depth, 1, spk), lambda n: (0, 0, 0)),
            pl.BlockSpec((3, 2 * 3 * size, 2 * chan), lambda n: (0, 0, 0)),
            pl.BlockSpec((1, 2 * chan), lambda n: (0, 0)),
        ],
        out_specs=pl.BlockSpec((_PAIRS, Hup, Wup, 2 * chan),
                               lambda n: (n, 0, 0, 0)),
        scratch_shapes=[
            pltpu.VMEM((_PAIRS, Hin + 2, Win + 2, cpk), bf16),
            pltpu.VMEM((_PAIRS, Hup + 2, Wup + 2, spk), bf16),
            pltpu.VMEM((_PAIRS, Hup + 2, Wup + 2, spk), bf16),
        ],
        compiler_params=pltpu.CompilerParams(
            dimension_semantics=("parallel",),
            vmem_limit_bytes=64 * 1024 * 1024),
    )(xp, s2, hw_f, hb_2, cw_f, cb_3, tw_f, tb_2)

    o = out.reshape(N // 2, Hup, Wup, 2, chan).transpose(0, 3, 4, 1, 2)
    return o.reshape(N, chan, Hup, Wup)


# speedup vs baseline: 2.4142x; 1.0180x over previous
import functools

import jax
import jax.numpy as jnp
from jax.experimental import pallas as pl
from jax.experimental.pallas import tpu as pltpu

_PAIRS = 2


def _conv3x3_packed(rows_all, w_full, bias_row, H, W, Cout, relu):
    taps = []
    for dy in range(3):
        rows = rows_all[dy:dy + H]
        left = jnp.concatenate(
            [rows[:, W + 1:W + 2, :], rows[:, 0:W - 1, :]], axis=1)
        taps += [left, rows[:, 0:W, :], rows[:, 1:W + 1, :]]
    patch = jnp.concatenate(taps, axis=-1)
    acc = jnp.broadcast_to(bias_row[None, :, :], (H, W, Cout)) + (
        jax.lax.dot_general(
            patch, w_full,
            dimension_numbers=(((2,), (0,)), ((), ())),
            preferred_element_type=jnp.float32))
    if relu:
        acc = jnp.maximum(acc, 0.0)
    return acc


def _upscaler_kernel(x_ref, s2_ref, hw_ref, hb_ref, cw_ref, cb_ref, tw_ref,
                     tb_ref, o_ref, hpad_ref, bufa_ref, bufb_ref, *,
                     Hin, Win, Hup, Wup, cpk, spk, chan, depth):
    bf16 = jnp.bfloat16

    for buf in (bufa_ref, bufb_ref):
        buf[:, 0:1, :, :] = jnp.zeros((_PAIRS, 1, Wup + 2, spk), bf16)
        buf[:, Hup + 1:Hup + 2, :, :] = jnp.zeros((_PAIRS, 1, Wup + 2, spk), bf16)
        buf[:, :, Wup:Wup + 2, :] = jnp.zeros((_PAIRS, Hup + 2, 2, spk), bf16)

    hpad_ref[...] = jnp.zeros((_PAIRS, Hin + 2, Win + 2, cpk), bf16)
    hpad_ref[:, 1:Hin + 1, 0:Win, :] = x_ref[...].astype(bf16)
    s2 = s2_ref[...]
    for p in range(_PAIRS):
        head = _conv3x3_packed(
            hpad_ref[p], hw_ref[...], hb_ref[...],
            Hin, Win, spk, relu=True)
        up = jax.lax.dot_general(
            s2, head.astype(bf16).reshape(Hin * Win, spk),
            dimension_numbers=(((1,), (0,)), ((), ())),
            preferred_element_type=jnp.float32)
        bufa_ref[p, 1:Hup + 1, 0:Wup, :] = (
            up.astype(bf16).reshape(Hup, Wup, spk))

    def core_pair(i, carry):
        l0 = 2 * i
        l1 = l0 + 1
        for p in range(_PAIRS):
            a0 = _conv3x3_packed(
                bufa_ref[p], cw_ref[l0], cb_ref[l0],
                Hup, Wup, spk, relu=True)
            bufb_ref[p, 1:Hup + 1, 0:Wup, :] = a0.astype(bf16)
        for p in range(_PAIRS):
            a1 = _conv3x3_packed(
                bufb_ref[p], cw_ref[l1], cb_ref[l1],
                Hup, Wup, spk, relu=True)
            bufa_ref[p, 1:Hup + 1, 0:Wup, :] = a1.astype(bf16)
        return carry

    jax.lax.fori_loop(0, depth // 2, core_pair, 0)

    for p in range(_PAIRS):
        tail = _conv3x3_packed(
            bufa_ref[p], tw_ref[...], tb_ref[...],
            Hup, Wup, 2 * chan, relu=False)
        o_ref[p, :, :, :] = tail.astype(o_ref.dtype)


def _block_diag2(w):
    K, C = w.shape[-2], w.shape[-1]
    z = jnp.zeros(w.shape[:-2] + (K, C), w.dtype)
    top = jnp.concatenate([w, z], axis=-1)
    bot = jnp.concatenate([z, w], axis=-1)
    return jnp.concatenate([top, bot], axis=-2)


def _fold_packed(w):
    kH, kW, Cin, Cout = w.shape
    wbd = _block_diag2(w)
    return wbd.reshape(kH * kW * 2 * Cin, 2 * Cout)


def kernel(x, hw, hb, cw, cb, tw, tb):
    N, chan, Hin, Win = x.shape
    size = hw.shape[-1]
    depth = cw.shape[0]
    fac = 2
    Hup, Wup = Hin * fac, Win * fac
    cpk, spk = 2 * chan, 2 * size
    bf16 = jnp.bfloat16

    xh = jnp.transpose(x, (0, 2, 3, 1))
    xp = xh.reshape(N // 2, 2, Hin, Win, chan).transpose(
        0, 2, 3, 1, 4).reshape(N // 2, Hin, Win, cpk)

    pos = jnp.arange(Hup * Wup)
    src = (pos // Wup // fac) * Win + (pos % Wup) // fac
    s2 = jax.nn.one_hot(src, Hin * Win, dtype=bf16)

    hw_f = _fold_packed(hw).astype(bf16)
    cw_f = jax.vmap(_fold_packed)(cw).astype(bf16)
    tw_f = _fold_packed(tw).astype(bf16)
    hb_2 = jnp.tile(hb, 2).reshape(1, spk)
    cb_3 = jnp.tile(cb, (1, 2)).reshape(depth, 1, spk)
    tb_2 = jnp.tile(tb, 2).reshape(1, 2 * chan)

    kern = functools.partial(_upscaler_kernel, Hin=Hin, Win=Win, Hup=Hup,
                             Wup=Wup, cpk=cpk, spk=spk, chan=chan, depth=depth)

    out = pl.pallas_call(
        kern,
        out_shape=jax.ShapeDtypeStruct((N // 2, Hup, Wup, 2 * chan), x.dtype),
        grid=(N // 2 // _PAIRS,),
        in_specs=[
            pl.BlockSpec((_PAIRS, Hin, Win, cpk), lambda n: (n, 0, 0, 0)),
            pl.BlockSpec((Hup * Wup, Hin * Win), lambda n: (0, 0)),
            pl.BlockSpec((9 * cpk, spk), lambda n: (0, 0)),
            pl.BlockSpec((1, spk), lambda n: (0, 0)),
            pl.BlockSpec((depth, 9 * spk, spk), lambda n: (0, 0, 0)),
            pl.BlockSpec((depth, 1, spk), lambda n: (0, 0, 0)),
            pl.BlockSpec((9 * spk, 2 * chan), lambda n: (0, 0)),
            pl.BlockSpec((1, 2 * chan), lambda n: (0, 0)),
        ],
        out_specs=pl.BlockSpec((_PAIRS, Hup, Wup, 2 * chan),
                               lambda n: (n, 0, 0, 0)),
        scratch_shapes=[
            pltpu.VMEM((_PAIRS, Hin + 2, Win + 2, cpk), bf16),
            pltpu.VMEM((_PAIRS, Hup + 2, Wup + 2, spk), bf16),
            pltpu.VMEM((_PAIRS, Hup + 2, Wup + 2, spk), bf16),
        ],
        compiler_params=pltpu.CompilerParams(
            dimension_semantics=("parallel",),
            vmem_limit_bytes=64 * 1024 * 1024),
    )(xp, s2, hw_f, hb_2, cw_f, cb_3, tw_f, tb_2)

    o = out.reshape(N // 2, Hup, Wup, 2, chan).transpose(0, 3, 4, 1, 2)
    return o.reshape(N, chan, Hup, Wup)
